# baseline (speedup 1.0000x reference)
import functools

import jax
import jax.numpy as jnp
from jax import lax
from jax.experimental import pallas as pl
from jax.experimental.pallas import tpu as pltpu
from jax.experimental.pallas import tpu_sc as plsc

_MAX_SPIKE = 100.0
_P = 0.1
_NC, _NS = 2, 16
_NW = _NC * _NS
_CW = 2048


def _spike_consts(B, T, dtype):
    key = jax.random.key(42)
    k1, k2, k3 = jax.random.split(key, 3)
    probas = jax.random.uniform(k1, (B,), dtype=jnp.float32)
    mask = probas > (1.0 - _P)
    pos = jax.random.randint(k2, (B,), 0, T - 2)
    sign = jnp.where(jax.random.randint(k3, (), 0, 2) == 0, -1.0, 1.0).astype(dtype)
    vals = jnp.where(mask, sign * _MAX_SPIKE, 0.0).astype(dtype)
    return pos, vals


def _sc_body(B, T, x_hbm, pos_hbm, val_hbm, out_hbm, posv, valv, buf0, buf1,
             sem_in, sem_out):
    rows = B // _NW
    ngrp = rows // 16
    wid = lax.axis_index("s") * _NC + lax.axis_index("c")
    r0 = wid * rows
    pltpu.sync_copy(pos_hbm.at[pl.ds(r0, rows)], posv)
    pltpu.sync_copy(val_hbm.at[pl.ds(r0, rows)], valv)
    lane = lax.broadcasted_iota(jnp.int32, (16,), 0)
    pos16 = [posv[pl.ds(g * 16, 16)] for g in range(ngrp)]
    val16 = [valv[pl.ds(g * 16, 16)] for g in range(ngrp)]
    bufs = (buf0, buf1)

    in_h = [None] * rows
    out_h = [None] * rows
    in_h[0] = pltpu.async_copy(x_hbm.at[r0], bufs[0], sem_in)
    for j in range(rows):
        if j + 1 < rows:
            if j - 1 >= 0:
                out_h[j - 1].wait()
            in_h[j + 1] = pltpu.async_copy(
                x_hbm.at[r0 + j + 1], bufs[(j + 1) % 2], sem_in)
        in_h[j].wait()
        g, l = divmod(j, 16)
        plsc.addupdate_scatter(bufs[j % 2], [pos16[g]], val16[g],
                               mask=lane == l)
        out_h[j] = pltpu.async_copy(bufs[j % 2], out_hbm.at[r0 + j], sem_out)
    out_h[rows - 2].wait()
    out_h[rows - 1].wait()


def kernel(x):
    B, T = x.shape
    pos, vals = _spike_consts(B, T, x.dtype)
    mesh = plsc.VectorSubcoreMesh(core_axis_name="c", subcore_axis_name="s",
                                  num_cores=_NC, num_subcores=_NS)
    rows = B // _NW
    sc_call = pl.kernel(
        functools.partial(_sc_body, B, T),
        out_type=jax.ShapeDtypeStruct((B, T), x.dtype),
        mesh=mesh,
        compiler_params=pltpu.CompilerParams(needs_layout_passes=False),
        scratch_types=[
            pltpu.VMEM((rows,), jnp.int32),
            pltpu.VMEM((rows,), jnp.float32),
            pltpu.VMEM((T,), jnp.float32),
            pltpu.VMEM((T,), jnp.float32),
            pltpu.SemaphoreType.DMA,
            pltpu.SemaphoreType.DMA,
        ],
    )
    return sc_call(x, pos, vals)
